# baseline (speedup 1.0000x reference)
import functools

import jax
import jax.numpy as jnp
from jax import lax
from jax.experimental import pallas as pl
from jax.experimental.pallas import tpu as pltpu
from jax.experimental.pallas import tpu_sc as plsc

_NC = 2
_NS = 16
_NW = _NC * _NS


@functools.partial(jax.jit, static_argnames=("chunk",))
def _sc_gather(idx, table, chunk):
    (b_total,) = idx.shape
    v, d = table.shape
    b_per_w = b_total // _NW
    n_chunks = b_per_w // chunk

    mesh = plsc.VectorSubcoreMesh(core_axis_name="c", subcore_axis_name="s")

    @functools.partial(
        pl.kernel,
        out_type=jax.ShapeDtypeStruct((b_total, d), jnp.float32),
        mesh=mesh,
        scratch_types=[
            pltpu.VMEM((chunk,), jnp.int32),
            pltpu.VMEM((chunk, d), jnp.float32),
            pltpu.SemaphoreType.DMA,
        ],
        compiler_params=pltpu.CompilerParams(use_tc_tiling_on_sc=False),
    )
    def k(idx_hbm, table_hbm, out_hbm, idx_v, rows_v, sem):
        wid = lax.axis_index("s") * _NC + lax.axis_index("c")
        base_w = wid * b_per_w

        def body(i, carry):
            base = base_w + i * chunk
            pltpu.sync_copy(idx_hbm.at[pl.ds(base, chunk)], idx_v)
            pltpu.async_copy(table_hbm.at[idx_v], rows_v, sem).wait()
            pltpu.sync_copy(rows_v, out_hbm.at[pl.ds(base, chunk)])
            return carry

        lax.fori_loop(0, n_chunks, body, 0)

    return k(idx, table)


def kernel(batch_seq, table):
    b, l = batch_seq.shape
    d = table.shape[1]
    idx = batch_seq.reshape(-1).astype(jnp.int32)
    out = _sc_gather(idx, table, chunk=6400)
    return out.reshape(b, l, d)


# speedup vs baseline: 1.0051x; 1.0051x over previous
import functools

import jax
import jax.numpy as jnp
from jax import lax
from jax.experimental import pallas as pl
from jax.experimental.pallas import tpu as pltpu
from jax.experimental.pallas import tpu_sc as plsc

_NC = 2
_NS = 16
_NW = _NC * _NS


@functools.partial(jax.jit, static_argnames=("chunk", "nbuf"))
def _sc_gather(idx, table, chunk, nbuf):
    (b_total,) = idx.shape
    v, d = table.shape
    b_per_w = b_total // _NW
    n_chunks = b_per_w // chunk

    mesh = plsc.VectorSubcoreMesh(core_axis_name="c", subcore_axis_name="s")

    scratch = [pltpu.VMEM((n_chunks, chunk), jnp.int32)]
    scratch += [pltpu.VMEM((chunk, d), jnp.float32) for _ in range(nbuf)]
    scratch += [pltpu.SemaphoreType.DMA for _ in range(2 * nbuf)]

    @functools.partial(
        pl.kernel,
        out_type=jax.ShapeDtypeStruct((b_total, d), jnp.float32),
        mesh=mesh,
        scratch_types=scratch,
        compiler_params=pltpu.CompilerParams(use_tc_tiling_on_sc=False),
    )
    def k(idx_hbm, table_hbm, out_hbm, idx_v, *bufs):
        rows = bufs[:nbuf]
        sem_g = bufs[nbuf:2 * nbuf]
        sem_s = bufs[2 * nbuf:]
        wid = lax.axis_index("s") * _NC + lax.axis_index("c")
        base_w = wid * b_per_w

        pltpu.sync_copy(idx_hbm.at[wid], idx_v)

        def gather(i):
            b = i % nbuf
            return pltpu.async_copy(table_hbm.at[idx_v.at[i]], rows[b], sem_g[b])

        def store(i):
            b = i % nbuf
            return pltpu.async_copy(
                rows[b], out_hbm.at[pl.ds(base_w + i * chunk, chunk)], sem_s[b])

        g = [gather(i) for i in range(min(nbuf, n_chunks))]
        s = [None] * nbuf
        for i in range(n_chunks):
            b = i % nbuf
            g[b].wait()
            s[b] = store(i)
            j = i + nbuf
            if j < n_chunks:
                s[b].wait()
                g[b] = gather(j)
        for i in range(max(0, n_chunks - nbuf), n_chunks):
            s[i % nbuf].wait()

    return k(idx.reshape(_NW, n_chunks, chunk), table)


def kernel(batch_seq, table):
    b, l = batch_seq.shape
    d = table.shape[1]
    idx = batch_seq.reshape(-1).astype(jnp.int32)
    out = _sc_gather(idx, table, chunk=1600, nbuf=4)
    return out.reshape(b, l, d)


# speedup vs baseline: 1.2303x; 1.2241x over previous
import functools

import jax
import jax.numpy as jnp
from jax import lax
from jax.experimental import pallas as pl
from jax.experimental.pallas import tpu as pltpu
from jax.experimental.pallas import tpu_sc as plsc

_NC = 2
_NS = 16
_NW = _NC * _NS

_B = 4096
_L = 200
_D = 16
_LC = 8
_CHUNK = _LC * 128
_NCHUNK = _L // _LC


@jax.jit
def _sc_gather(idx3, table):
    mesh = plsc.VectorSubcoreMesh(core_axis_name="c", subcore_axis_name="s")

    @functools.partial(
        pl.kernel,
        out_type=jax.ShapeDtypeStruct((_L, 2, _NW, 1024), jnp.float32),
        mesh=mesh,
        scratch_types=[
            pltpu.VMEM((_NCHUNK, _CHUNK), jnp.int32),
            pltpu.VMEM((_CHUNK, _D), jnp.float32),
            pltpu.VMEM((_LC, 2, 1024), jnp.float32),
            pltpu.SemaphoreType.DMA,
        ],
        compiler_params=pltpu.CompilerParams(
            use_tc_tiling_on_sc=False, needs_layout_passes=False),
    )
    def k(idx_hbm, table_hbm, out_hbm, idx_v, rows_v, trans_v, sem):
        wid = lax.axis_index("s") * _NC + lax.axis_index("c")
        pltpu.sync_copy(idx_hbm.at[:, wid], idx_v)

        iota = lax.iota(jnp.int32, 16)
        d1v = iota >> 3
        innerv = (iota & 7) * 128

        def chunk_body(i, carry):
            pltpu.async_copy(table_hbm.at[idx_v.at[i]], rows_v, sem).wait()
            for l_i in range(_LC):
                lv = jnp.full((16,), l_i, jnp.int32)

                def row_group(g, c):
                    for u in range(8):
                        b = g * 8 + u
                        row = rows_v[l_i * 128 + b]
                        plsc.store_scatter(trans_v, [lv, d1v, innerv + b], row)
                    return c

                lax.fori_loop(0, 16, row_group, 0)
            pltpu.sync_copy(trans_v, out_hbm.at[pl.ds(i * _LC, _LC), :, wid])
            return carry

        lax.fori_loop(0, _NCHUNK, chunk_body, 0)

    return k(idx3, table)


def kernel(batch_seq, table):
    idx3 = (batch_seq.reshape(_NW, 128, _L // 8, 8).transpose(2, 0, 3, 1)
            .reshape(_L // 8, _NW, 1024).astype(jnp.int32))
    out5 = _sc_gather(idx3, table)
    out = (out5.reshape(_L, 2, _NW, 8, 128).transpose(2, 4, 0, 1, 3)
           .reshape(_B, _L, _D))
    return out


# speedup vs baseline: 1.6905x; 1.3740x over previous
import functools

import jax
import jax.numpy as jnp
from jax import lax
from jax.experimental import pallas as pl
from jax.experimental.pallas import tpu as pltpu
from jax.experimental.pallas import tpu_sc as plsc

_NC = 2
_NS = 16
_NW = _NC * _NS

_B = 4096
_L = 200
_D = 16
_LC = 8
_CHUNK = _LC * 128
_NCHUNK = _L // _LC


@jax.jit
def _sc_gather(idx3, table):
    mesh = plsc.VectorSubcoreMesh(core_axis_name="c", subcore_axis_name="s")

    @functools.partial(
        pl.kernel,
        out_type=jax.ShapeDtypeStruct((_L, 2, _NW, 8, 128), jnp.float32),
        mesh=mesh,
        scratch_types=[
            pltpu.VMEM((_NCHUNK, _CHUNK), jnp.int32),
            pltpu.VMEM((_CHUNK, _D), jnp.float32),
            pltpu.VMEM((_CHUNK, _D), jnp.float32),
            pltpu.VMEM((_LC, 2, 8, 129), jnp.float32),
            pltpu.VMEM((_LC, 2, 8, 129), jnp.float32),
            pltpu.SemaphoreType.DMA,
            pltpu.SemaphoreType.DMA,
            pltpu.SemaphoreType.DMA,
            pltpu.SemaphoreType.DMA,
        ],
        compiler_params=pltpu.CompilerParams(
            use_tc_tiling_on_sc=False, needs_layout_passes=False),
    )
    def k(idx_hbm, table_hbm, out_hbm, idx_v, rows0, rows1, tr0, tr1,
          sg0, sg1, ss0, ss1):
        rows = (rows0, rows1)
        trans = (tr0, tr1)
        sem_g = (sg0, sg1)
        sem_s = (ss0, ss1)
        wid = lax.axis_index("s") * _NC + lax.axis_index("c")
        pltpu.sync_copy(idx_hbm.at[:, wid], idx_v)

        iota = lax.iota(jnp.int32, 16)
        d1v = iota >> 3
        d2v = iota & 7

        def gather_cp(i, u):
            return pltpu.make_async_copy(
                table_hbm.at[idx_v.at[i]], rows[u], sem_g[u])

        def store_cp(i, u):
            return pltpu.make_async_copy(
                trans[u].at[:, :, :, pl.ds(0, 128)],
                out_hbm.at[pl.ds(i * _LC, _LC), :, wid], sem_s[u])

        def transpose(u):
            rv, tv = rows[u], trans[u]
            for l_i in range(_LC):
                lv = jnp.full((16,), l_i, jnp.int32)

                def row_group(g, c):
                    for s in range(8):
                        b = g * 8 + s
                        row = rv[l_i * 128 + b]
                        bv = jnp.full((16,), 0, jnp.int32) + b
                        plsc.store_scatter(tv, [lv, d1v, d2v, bv], row)
                    return c

                lax.fori_loop(0, 16, row_group, 0)

        gather_cp(0, 0).start()
        gather_cp(1, 1).start()

        def body(j, carry):
            for u in (0, 1):
                i = 2 * j + u
                gather_cp(i, u).wait()

                @pl.when(j > 0)
                def _():
                    store_cp(i - 2, u).wait()

                transpose(u)
                store_cp(i, u).start()

                @pl.when(i + 2 < _NCHUNK)
                def _():
                    gather_cp(i + 2, u).start()

            return carry

        lax.fori_loop(0, (_NCHUNK - 1) // 2, body, 0)

        last = _NCHUNK - 1
        gather_cp(last, 0).wait()
        store_cp(last - 2, 0).wait()
        transpose(0)
        store_cp(last, 0).start()
        store_cp(last - 1, 1).wait()
        store_cp(last, 0).wait()

    return k(idx3, table)


def kernel(batch_seq, table):
    idx3 = (batch_seq.reshape(_NW, 128, _L // 8, 8).transpose(2, 0, 3, 1)
            .reshape(_L // 8, _NW, 1024).astype(jnp.int32))
    out5 = _sc_gather(idx3, table)
    return out5.transpose(2, 4, 0, 1, 3).reshape(_B, _L, _D)


# speedup vs baseline: 1.7929x; 1.0606x over previous
import functools

import jax
import jax.numpy as jnp
from jax import lax
from jax.experimental import pallas as pl
from jax.experimental.pallas import tpu as pltpu
from jax.experimental.pallas import tpu_sc as plsc

_NC = 2
_NS = 16
_NW = _NC * _NS

_B = 4096
_L = 200
_D = 16
_LC = 8
_CHUNK = _LC * 128
_NCHUNK = _L // _LC


_V = 1000000
_NT = 7813
_FULL_T = 7812


@jax.jit
def _sc_detile(tab_t):
    mesh = plsc.VectorSubcoreMesh(core_axis_name="c", subcore_axis_name="s")

    @functools.partial(
        pl.kernel,
        out_type=jax.ShapeDtypeStruct((_V // 64, 8, 128), jnp.float32),
        mesh=mesh,
        scratch_types=[
            pltpu.VMEM((8, 131), jnp.float32),
            pltpu.VMEM((8, 131), jnp.float32),
            pltpu.VMEM((8, 131), jnp.float32),
            pltpu.VMEM((8, 131), jnp.float32),
            pltpu.VMEM((2, 8, 128), jnp.float32),
            pltpu.VMEM((2, 8, 128), jnp.float32),
            pltpu.SemaphoreType.DMA,
            pltpu.SemaphoreType.DMA,
            pltpu.SemaphoreType.DMA,
            pltpu.SemaphoreType.DMA,
        ],
        compiler_params=pltpu.CompilerParams(
            use_tc_tiling_on_sc=True, needs_layout_passes=False),
    )
    def k(tab_hbm, out_hbm, xa0, xb0, xa1, xb1, y0, y1, sg0, sg1, ss0, ss1):
        xa = (xa0, xa1)
        xb = (xb0, xb1)
        y = (y0, y1)
        sem_g = (sg0, sg1)
        sem_s = (ss0, ss1)
        wid = lax.axis_index("s") * _NC + lax.axis_index("c")
        iota = lax.iota(jnp.int32, 16)
        d2v = iota & 7
        lov = iota < 8

        base = wid * 244 + jnp.minimum(wid, 4)
        cnt = jnp.where(wid < 4, 245, 244)

        def in_cp(t, u):
            a = pltpu.make_async_copy(
                tab_hbm.at[pl.ds(0, 8), pl.ds(t * 128, 128)],
                xa[u].at[:, pl.ds(0, 128)], sem_g[u])
            b = pltpu.make_async_copy(
                tab_hbm.at[pl.ds(8, 8), pl.ds(t * 128, 128)],
                xb[u].at[:, pl.ds(0, 128)], sem_g[u])
            return a, b

        def out_cp(t, u, nrow=2):
            return pltpu.make_async_copy(
                y[u].at[pl.ds(0, nrow)], out_hbm.at[pl.ds(t * 2, nrow)],
                sem_s[u])

        def transpose(u):
            def v_body(g, c):
                for s in range(8):
                    v2 = g * 8 + s
                    vv = jnp.full((16,), 0, jnp.int32) + v2
                    av = plsc.load_gather(xa[u], [d2v, vv], mask=lov)
                    bv = plsc.load_gather(xb[u], [d2v, vv], mask=~lov)
                    row = jnp.where(lov, av, bv)
                    y[u].at[v2 // 64, (v2 % 64) // 8][
                        pl.ds((v2 % 8) * 16, 16)] = row
                return c

            lax.fori_loop(0, 16, v_body, 0)

        def start(t, u):
            a, b = in_cp(t, u)
            a.start()
            b.start()

        def wait_in(t, u):
            a, b = in_cp(t, u)
            a.wait()
            b.wait()

        start(base, 0)
        start(base + 1, 1)

        def body(j, carry):
            for u in (0, 1):
                t = base + 2 * j + u
                wait_in(t, u)

                @pl.when(j > 0)
                def _():
                    out_cp(t - 2, u).wait()

                transpose(u)
                out_cp(t, u).start()

                @pl.when(2 * j + u + 2 < cnt)
                def _():
                    start(t + 2, u)

            return carry

        lax.fori_loop(0, 122, body, 0)

        @pl.when(cnt == 245)
        def _():
            t = base + 244
            wait_in(t, 0)
            out_cp(t - 2, 0).wait()
            transpose(0)
            out_cp(t, 0).start()

        @pl.when(cnt == 244)
        def _():
            out_cp(base + 242, 0).wait()
            out_cp(base + 243, 1).wait()

        @pl.when(cnt == 245)
        def _():
            out_cp(base + 243, 1).wait()
            out_cp(base + 244, 0).wait()

        @pl.when(wid == _NW - 1)
        def _():
            t = jnp.int32(_FULL_T)
            start(t, 0)
            wait_in(t, 0)
            transpose(0)
            out_cp(t, 0, nrow=1).start()
            out_cp(t, 0, nrow=1).wait()

    return k(tab_t)


@jax.jit
def _sc_gather(idx3, table):
    mesh = plsc.VectorSubcoreMesh(core_axis_name="c", subcore_axis_name="s")

    @functools.partial(
        pl.kernel,
        out_type=jax.ShapeDtypeStruct((_L, 2, _NW, 8, 128), jnp.float32),
        mesh=mesh,
        scratch_types=[
            pltpu.VMEM((_NCHUNK, _CHUNK), jnp.int32),
            pltpu.VMEM((_CHUNK, _D), jnp.float32),
            pltpu.VMEM((_CHUNK, _D), jnp.float32),
            pltpu.VMEM((_LC, 2, 8, 129), jnp.float32),
            pltpu.VMEM((_LC, 2, 8, 129), jnp.float32),
            pltpu.SemaphoreType.DMA,
            pltpu.SemaphoreType.DMA,
            pltpu.SemaphoreType.DMA,
            pltpu.SemaphoreType.DMA,
        ],
        compiler_params=pltpu.CompilerParams(
            use_tc_tiling_on_sc=False, needs_layout_passes=False),
    )
    def k(idx_hbm, table_hbm, out_hbm, idx_v, rows0, rows1, tr0, tr1,
          sg0, sg1, ss0, ss1):
        rows = (rows0, rows1)
        trans = (tr0, tr1)
        sem_g = (sg0, sg1)
        sem_s = (ss0, ss1)
        wid = lax.axis_index("s") * _NC + lax.axis_index("c")
        pltpu.sync_copy(idx_hbm.at[:, wid], idx_v)

        iota = lax.iota(jnp.int32, 16)
        d1v = iota >> 3
        d2v = iota & 7

        def gather_cp(i, u):
            return pltpu.make_async_copy(
                table_hbm.at[idx_v.at[i]], rows[u], sem_g[u])

        def store_cp(i, u):
            return pltpu.make_async_copy(
                trans[u].at[:, :, :, pl.ds(0, 128)],
                out_hbm.at[pl.ds(i * _LC, _LC), :, wid], sem_s[u])

        def transpose(u):
            rv, tv = rows[u], trans[u]
            for l_i in range(_LC):
                lv = jnp.full((16,), l_i, jnp.int32)

                def row_group(g, c):
                    for s in range(8):
                        b = g * 8 + s
                        row = rv[l_i * 128 + b]
                        bv = jnp.full((16,), 0, jnp.int32) + b
                        plsc.store_scatter(tv, [lv, d1v, d2v, bv], row)
                    return c

                lax.fori_loop(0, 16, row_group, 0)

        gather_cp(0, 0).start()
        gather_cp(1, 1).start()

        def body(j, carry):
            for u in (0, 1):
                i = 2 * j + u
                gather_cp(i, u).wait()

                @pl.when(j > 0)
                def _():
                    store_cp(i - 2, u).wait()

                transpose(u)
                store_cp(i, u).start()

                @pl.when(i + 2 < _NCHUNK)
                def _():
                    gather_cp(i + 2, u).start()

            return carry

        lax.fori_loop(0, (_NCHUNK - 1) // 2, body, 0)

        last = _NCHUNK - 1
        gather_cp(last, 0).wait()
        store_cp(last - 2, 0).wait()
        transpose(0)
        store_cp(last, 0).start()
        store_cp(last - 1, 1).wait()
        store_cp(last, 0).wait()

    return k(idx3, table)


def kernel(batch_seq, table):
    idx3 = (batch_seq.reshape(_NW, 128, _L // 8, 8).transpose(2, 0, 3, 1)
            .reshape(_L // 8, _NW, 1024).astype(jnp.int32))
    tab_lin = _sc_detile(table.T).reshape(_V, _D)
    out5 = _sc_gather(idx3, tab_lin)
    return out5.transpose(2, 4, 0, 1, 3).reshape(_B, _L, _D)


# speedup vs baseline: 1.8475x; 1.0304x over previous
import functools

import jax
import jax.numpy as jnp
from jax import lax
from jax.experimental import pallas as pl
from jax.experimental.pallas import tpu as pltpu
from jax.experimental.pallas import tpu_sc as plsc

_NC = 2
_NS = 16
_NW = _NC * _NS

_B = 4096
_L = 200
_D = 16
_LC = 8
_CHUNK = _LC * 128
_NCHUNK = _L // _LC


_V = 1000000
_NT = 7813
_FULL_T = 7812


@jax.jit
def _sc_detile(tab_t):
    mesh = plsc.VectorSubcoreMesh(core_axis_name="c", subcore_axis_name="s")
    mw = 512
    pitch = 515

    @functools.partial(
        pl.kernel,
        out_type=jax.ShapeDtypeStruct((_V // 64, 8, 128), jnp.float32),
        mesh=mesh,
        scratch_types=[
            pltpu.VMEM((16, pitch), jnp.float32),
            pltpu.VMEM((16, pitch), jnp.float32),
            pltpu.VMEM((8, 8, 128), jnp.float32),
            pltpu.VMEM((8, 8, 128), jnp.float32),
            pltpu.SemaphoreType.DMA,
            pltpu.SemaphoreType.DMA,
            pltpu.SemaphoreType.DMA,
            pltpu.SemaphoreType.DMA,
        ],
        compiler_params=pltpu.CompilerParams(
            use_tc_tiling_on_sc=True, needs_layout_passes=False),
    )
    def k(tab_hbm, out_hbm, x0, x1, y0, y1, sg0, sg1, ss0, ss1):
        x = (x0, x1)
        y = (y0, y1)
        sem_g = (sg0, sg1)
        sem_s = (ss0, ss1)
        wid = lax.axis_index("s") * _NC + lax.axis_index("c")
        iota = lax.iota(jnp.int32, 16)

        base = wid * 61 + jnp.minimum(wid, 1)
        cnt = jnp.where(wid < 1, 62, 61)

        def in_cp(t, u, ncol=mw):
            a = pltpu.make_async_copy(
                tab_hbm.at[pl.ds(0, 8), pl.ds(t * mw, ncol)],
                x[u].at[pl.ds(0, 8), pl.ds(0, ncol)], sem_g[u])
            b = pltpu.make_async_copy(
                tab_hbm.at[pl.ds(8, 8), pl.ds(t * mw, ncol)],
                x[u].at[pl.ds(8, 8), pl.ds(0, ncol)], sem_g[u])
            return a, b

        def out_cp(t, u, nrow=8):
            return pltpu.make_async_copy(
                y[u].at[pl.ds(0, nrow)], out_hbm.at[pl.ds(t * 8, nrow)],
                sem_s[u])

        def transpose(u, ngroup=64):
            xv, yv = x[u], y[u]

            def g_body(g, c):
                row_ref = yv.at[g // 8, g % 8]
                for s in range(8):
                    vv = jnp.full((16,), 0, jnp.int32) + (g * 8 + s)
                    row_ref[pl.ds(s * 16, 16)] = plsc.load_gather(
                        xv, [iota, vv])
                return c

            lax.fori_loop(0, ngroup, g_body, 0)

        def start(t, u):
            a, b = in_cp(t, u)
            a.start()
            b.start()

        def wait_in(t, u):
            a, b = in_cp(t, u)
            a.wait()
            b.wait()

        start(base, 0)
        start(base + 1, 1)

        def body(j, carry):
            for u in (0, 1):
                t = base + 2 * j + u
                wait_in(t, u)

                @pl.when(j > 0)
                def _():
                    out_cp(t - 2, u).wait()

                transpose(u)
                out_cp(t, u).start()

                @pl.when(2 * j + u + 2 < cnt)
                def _():
                    start(t + 2, u)

            return carry

        lax.fori_loop(0, 30, body, 0)

        t60 = base + 60
        wait_in(t60, 0)
        out_cp(t60 - 2, 0).wait()
        transpose(0)
        out_cp(t60, 0).start()

        @pl.when(cnt == 62)
        def _():
            t61 = base + 61
            wait_in(t61, 1)
            out_cp(t61 - 2, 1).wait()
            transpose(1)
            out_cp(t61, 1).start()
            out_cp(t61, 1).wait()

        @pl.when(cnt == 61)
        def _():
            out_cp(base + 59, 1).wait()

        out_cp(t60, 0).wait()

        @pl.when(wid == _NW - 1)
        def _():
            tl = jnp.int32(_FULL_T)
            a = pltpu.make_async_copy(
                tab_hbm.at[pl.ds(0, 8), pl.ds(tl * 128, 128)],
                x[0].at[pl.ds(0, 8), pl.ds(0, 128)], sem_g[0])
            b = pltpu.make_async_copy(
                tab_hbm.at[pl.ds(8, 8), pl.ds(tl * 128, 128)],
                x[0].at[pl.ds(8, 8), pl.ds(0, 128)], sem_g[0])
            a.start()
            b.start()
            a.wait()
            b.wait()
            transpose(0, ngroup=8)
            fin = pltpu.make_async_copy(
                y[0].at[pl.ds(0, 1)],
                out_hbm.at[pl.ds(_FULL_T * 2, 1)], sem_s[0])
            fin.start()
            fin.wait()

    return k(tab_t)


@jax.jit
def _sc_gather(idx3, table):
    mesh = plsc.VectorSubcoreMesh(core_axis_name="c", subcore_axis_name="s")

    @functools.partial(
        pl.kernel,
        out_type=jax.ShapeDtypeStruct((_L, 2, _NW, 8, 128), jnp.float32),
        mesh=mesh,
        scratch_types=[
            pltpu.VMEM((_NCHUNK, _CHUNK), jnp.int32),
            pltpu.VMEM((_CHUNK, _D), jnp.float32),
            pltpu.VMEM((_CHUNK, _D), jnp.float32),
            pltpu.VMEM((_LC, 2, 8, 129), jnp.float32),
            pltpu.VMEM((_LC, 2, 8, 129), jnp.float32),
            pltpu.SemaphoreType.DMA,
            pltpu.SemaphoreType.DMA,
            pltpu.SemaphoreType.DMA,
            pltpu.SemaphoreType.DMA,
        ],
        compiler_params=pltpu.CompilerParams(
            use_tc_tiling_on_sc=False, needs_layout_passes=False),
    )
    def k(idx_hbm, table_hbm, out_hbm, idx_v, rows0, rows1, tr0, tr1,
          sg0, sg1, ss0, ss1):
        rows = (rows0, rows1)
        trans = (tr0, tr1)
        sem_g = (sg0, sg1)
        sem_s = (ss0, ss1)
        wid = lax.axis_index("s") * _NC + lax.axis_index("c")
        pltpu.sync_copy(idx_hbm.at[:, wid], idx_v)

        iota = lax.iota(jnp.int32, 16)
        d1v = iota >> 3
        d2v = iota & 7

        def gather_cp(i, u):
            return pltpu.make_async_copy(
                table_hbm.at[idx_v.at[i]], rows[u], sem_g[u])

        def store_cp(i, u):
            return pltpu.make_async_copy(
                trans[u].at[:, :, :, pl.ds(0, 128)],
                out_hbm.at[pl.ds(i * _LC, _LC), :, wid], sem_s[u])

        def transpose(u):
            rv, tv = rows[u], trans[u]
            for l_i in range(_LC):
                lv = jnp.full((16,), l_i, jnp.int32)

                def row_group(g, c):
                    for s in range(8):
                        b = g * 8 + s
                        row = rv[l_i * 128 + b]
                        bv = jnp.full((16,), 0, jnp.int32) + b
                        plsc.store_scatter(tv, [lv, d1v, d2v, bv], row)
                    return c

                lax.fori_loop(0, 16, row_group, 0)

        gather_cp(0, 0).start()
        gather_cp(1, 1).start()

        def body(j, carry):
            for u in (0, 1):
                i = 2 * j + u
                gather_cp(i, u).wait()

                @pl.when(j > 0)
                def _():
                    store_cp(i - 2, u).wait()

                transpose(u)
                store_cp(i, u).start()

                @pl.when(i + 2 < _NCHUNK)
                def _():
                    gather_cp(i + 2, u).start()

            return carry

        lax.fori_loop(0, (_NCHUNK - 1) // 2, body, 0)

        last = _NCHUNK - 1
        gather_cp(last, 0).wait()
        store_cp(last - 2, 0).wait()
        transpose(0)
        store_cp(last, 0).start()
        store_cp(last - 1, 1).wait()
        store_cp(last, 0).wait()

    return k(idx3, table)


def kernel(batch_seq, table):
    idx3 = (batch_seq.reshape(_NW, 128, _L // 8, 8).transpose(2, 0, 3, 1)
            .reshape(_L // 8, _NW, 1024).astype(jnp.int32))
    tab_lin = _sc_detile(table.T).reshape(_V, _D)
    out5 = _sc_gather(idx3, tab_lin)
    return out5.transpose(2, 4, 0, 1, 3).reshape(_B, _L, _D)


# speedup vs baseline: 2.4523x; 1.3274x over previous
import functools

import jax
import jax.numpy as jnp
from jax import lax
from jax.experimental import pallas as pl
from jax.experimental.pallas import tpu as pltpu
from jax.experimental.pallas import tpu_sc as plsc

_NC = 2
_NS = 16
_NW = _NC * _NS

_B = 4096
_L = 200
_D = 16
_LC = 8
_CHUNK = _LC * 128
_NCHUNK = _L // _LC


_V = 1000000
_NT = 7813
_FULL_T = 7812


@jax.jit
def _sc_detile(tab_t):
    mesh = plsc.VectorSubcoreMesh(core_axis_name="c", subcore_axis_name="s")
    mw = 512
    pitch = 515

    @functools.partial(
        pl.kernel,
        out_type=jax.ShapeDtypeStruct((_V // 64, 8, 128), jnp.float32),
        mesh=mesh,
        scratch_types=[
            pltpu.VMEM((16, pitch), jnp.float32),
            pltpu.VMEM((16, pitch), jnp.float32),
            pltpu.VMEM((8, 8, 128), jnp.float32),
            pltpu.VMEM((8, 8, 128), jnp.float32),
            pltpu.VMEM((8720,), jnp.float32),
            pltpu.VMEM((8720,), jnp.float32),
            pltpu.SemaphoreType.DMA,
            pltpu.SemaphoreType.DMA,
            pltpu.SemaphoreType.DMA,
            pltpu.SemaphoreType.DMA,
        ],
        compiler_params=pltpu.CompilerParams(
            use_tc_tiling_on_sc=True, needs_layout_passes=False),
    )
    def k(tab_hbm, out_hbm, x0, x1, y0, y1, sc0, sc1, sg0, sg1, ss0, ss1):
        x = (x0, x1)
        y = (y0, y1)
        scr = (sc0, sc1)
        sem_g = (sg0, sg1)
        sem_s = (ss0, ss1)
        wid = lax.axis_index("s") * _NC + lax.axis_index("c")
        iota = lax.iota(jnp.int32, 16)

        base = wid * 61 + jnp.minimum(wid, 1)
        cnt = jnp.where(wid < 1, 62, 61)

        def in_cp(t, u, ncol=mw):
            a = pltpu.make_async_copy(
                tab_hbm.at[pl.ds(0, 8), pl.ds(t * mw, ncol)],
                x[u].at[pl.ds(0, 8), pl.ds(0, ncol)], sem_g[u])
            b = pltpu.make_async_copy(
                tab_hbm.at[pl.ds(8, 8), pl.ds(t * mw, ncol)],
                x[u].at[pl.ds(8, 8), pl.ds(0, ncol)], sem_g[u])
            return a, b

        def out_cp(t, u, nrow=8):
            return pltpu.make_async_copy(
                y[u].at[pl.ds(0, nrow)], out_hbm.at[pl.ds(t * 8, nrow)],
                sem_s[u])

        def transpose(u, ngroup=64):
            xv, yv, sv = x[u], y[u], scr[u]
            pitch17 = iota * 17

            for d in range(16):

                def s_body(g, c, d=d):
                    vec = xv[d, pl.ds(g * 16, 16)]
                    plsc.store_scatter(sv, [pitch17 + (g * 272 + d)], vec)
                    return c

                lax.fori_loop(0, ngroup // 2, s_body, 0)

            def c_body(g, c):
                row_ref = yv.at[g // 8, g % 8]
                for s in range(8):
                    v2 = g * 8 + s
                    row_ref[pl.ds(s * 16, 16)] = sv[pl.ds(v2 * 17, 16)]
                return c

            lax.fori_loop(0, ngroup, c_body, 0)

        def start(t, u):
            a, b = in_cp(t, u)
            a.start()
            b.start()

        def wait_in(t, u):
            a, b = in_cp(t, u)
            a.wait()
            b.wait()

        start(base, 0)
        start(base + 1, 1)

        def body(j, carry):
            for u in (0, 1):
                t = base + 2 * j + u
                wait_in(t, u)

                @pl.when(j > 0)
                def _():
                    out_cp(t - 2, u).wait()

                transpose(u)
                out_cp(t, u).start()

                @pl.when(2 * j + u + 2 < cnt)
                def _():
                    start(t + 2, u)

            return carry

        lax.fori_loop(0, 30, body, 0)

        t60 = base + 60
        wait_in(t60, 0)
        out_cp(t60 - 2, 0).wait()
        transpose(0)
        out_cp(t60, 0).start()

        @pl.when(cnt == 62)
        def _():
            t61 = base + 61
            wait_in(t61, 1)
            out_cp(t61 - 2, 1).wait()
            transpose(1)
            out_cp(t61, 1).start()
            out_cp(t61, 1).wait()

        @pl.when(cnt == 61)
        def _():
            out_cp(base + 59, 1).wait()

        out_cp(t60, 0).wait()

        @pl.when(wid == _NW - 1)
        def _():
            tl = jnp.int32(_FULL_T)
            a = pltpu.make_async_copy(
                tab_hbm.at[pl.ds(0, 8), pl.ds(tl * 128, 128)],
                x[0].at[pl.ds(0, 8), pl.ds(0, 128)], sem_g[0])
            b = pltpu.make_async_copy(
                tab_hbm.at[pl.ds(8, 8), pl.ds(tl * 128, 128)],
                x[0].at[pl.ds(8, 8), pl.ds(0, 128)], sem_g[0])
            a.start()
            b.start()
            a.wait()
            b.wait()
            transpose(0, ngroup=8)
            fin = pltpu.make_async_copy(
                y[0].at[pl.ds(0, 1)],
                out_hbm.at[pl.ds(_FULL_T * 2, 1)], sem_s[0])
            fin.start()
            fin.wait()

    return k(tab_t)


@jax.jit
def _sc_gather(idx3, table):
    mesh = plsc.VectorSubcoreMesh(core_axis_name="c", subcore_axis_name="s")

    @functools.partial(
        pl.kernel,
        out_type=jax.ShapeDtypeStruct((_L, 2, _NW, 8, 128), jnp.float32),
        mesh=mesh,
        scratch_types=[
            pltpu.VMEM((_NCHUNK, _CHUNK), jnp.int32),
            pltpu.VMEM((_CHUNK, _D), jnp.float32),
            pltpu.VMEM((_CHUNK, _D), jnp.float32),
            pltpu.VMEM((_LC, 2, 8, 129), jnp.float32),
            pltpu.VMEM((_LC, 2, 8, 129), jnp.float32),
            pltpu.SemaphoreType.DMA,
            pltpu.SemaphoreType.DMA,
            pltpu.SemaphoreType.DMA,
            pltpu.SemaphoreType.DMA,
        ],
        compiler_params=pltpu.CompilerParams(
            use_tc_tiling_on_sc=False, needs_layout_passes=False),
    )
    def k(idx_hbm, table_hbm, out_hbm, idx_v, rows0, rows1, tr0, tr1,
          sg0, sg1, ss0, ss1):
        rows = (rows0, rows1)
        trans = (tr0, tr1)
        sem_g = (sg0, sg1)
        sem_s = (ss0, ss1)
        wid = lax.axis_index("s") * _NC + lax.axis_index("c")
        pltpu.sync_copy(idx_hbm.at[:, wid], idx_v)

        iota = lax.iota(jnp.int32, 16)
        d1v = iota >> 3
        d2v = iota & 7

        def gather_cp(i, u):
            return pltpu.make_async_copy(
                table_hbm.at[idx_v.at[i]], rows[u], sem_g[u])

        def store_cp(i, u):
            return pltpu.make_async_copy(
                trans[u].at[:, :, :, pl.ds(0, 128)],
                out_hbm.at[pl.ds(i * _LC, _LC), :, wid], sem_s[u])

        def transpose(u):
            rv, tv = rows[u], trans[u]
            for l_i in range(_LC):
                lv = jnp.full((16,), l_i, jnp.int32)

                def row_group(g, c):
                    for s in range(8):
                        b = g * 8 + s
                        row = rv[l_i * 128 + b]
                        bv = jnp.full((16,), 0, jnp.int32) + b
                        plsc.store_scatter(tv, [lv, d1v, d2v, bv], row)
                    return c

                lax.fori_loop(0, 16, row_group, 0)

        gather_cp(0, 0).start()
        gather_cp(1, 1).start()

        def body(j, carry):
            for u in (0, 1):
                i = 2 * j + u
                gather_cp(i, u).wait()

                @pl.when(j > 0)
                def _():
                    store_cp(i - 2, u).wait()

                transpose(u)
                store_cp(i, u).start()

                @pl.when(i + 2 < _NCHUNK)
                def _():
                    gather_cp(i + 2, u).start()

            return carry

        lax.fori_loop(0, (_NCHUNK - 1) // 2, body, 0)

        last = _NCHUNK - 1
        gather_cp(last, 0).wait()
        store_cp(last - 2, 0).wait()
        transpose(0)
        store_cp(last, 0).start()
        store_cp(last - 1, 1).wait()
        store_cp(last, 0).wait()

    return k(idx3, table)


def kernel(batch_seq, table):
    idx3 = (batch_seq.reshape(_NW, 128, _L // 8, 8).transpose(2, 0, 3, 1)
            .reshape(_L // 8, _NW, 1024).astype(jnp.int32))
    tab_lin = _sc_detile(table.T).reshape(_V, _D)
    out5 = _sc_gather(idx3, tab_lin)
    return out5.transpose(2, 4, 0, 1, 3).reshape(_B, _L, _D)
